# baseline (speedup 1.0000x reference)
import functools

import jax
import jax.numpy as jnp
from jax import lax
from jax.experimental import pallas as pl
from jax.experimental.pallas import tpu as pltpu
from jax.experimental.pallas import tpu_sc as plsc

B, N, D = 4, 2048, 1024
H, DH = 16, 64
C = N // 2
BC = B * C
BN = B * N
CCHUNK = 256


def _cumsum_col(v):
    n = v.shape[0]
    s = 1
    while s < n:
        shifted = jnp.concatenate(
            [jnp.zeros((s, 1), jnp.float32), lax.slice(v, (0, 0), (n - s, 1))],
            axis=0)
        v = v + shifted
        s *= 2
    return v


def _router_topk_body(x_ref, w_ref, topi_ref, inv_ref, gval_ref, zsum_ref):
    b = pl.program_id(0)
    x = x_ref[0]
    w = w_ref[...]
    logits = lax.dot_general(x, w, (((1,), (0,)), ((), ())),
                             preferred_element_type=jnp.float32)
    zsum_ref[...] = jnp.sum(logits * logits, keepdims=True).reshape(1, 1, 1)
    gate = jax.nn.sigmoid(logits)

    lc = jnp.where(logits == 0.0, 0.0, logits)
    u = lax.bitcast_convert_type(lc, jnp.uint32)
    mask = jnp.where((u >> jnp.uint32(31)) > jnp.uint32(0),
                     jnp.uint32(0xFFFFFFFF), jnp.uint32(0x80000000))
    key = u ^ mask

    t = jnp.zeros((1, 1), jnp.uint32)
    cf = jnp.float32(C)
    for bit in range(31, -1, -1):
        cand = t | jnp.uint32(1 << bit)
        cnt = jnp.sum((key >= cand).astype(jnp.float32))
        t = jnp.where(cnt >= cf, cand, t)

    gt = key > t
    eq = key == t
    cnt_gt = jnp.sum(gt.astype(jnp.float32))
    need = cf - cnt_gt
    eqf = eq.astype(jnp.float32)
    eq_excl = _cumsum_col(eqf) - eqf
    sel = jnp.logical_or(gt, jnp.logical_and(eq, eq_excl < need))
    self_f = sel.astype(jnp.float32)
    cum_incl = _cumsum_col(self_f)
    slot = cum_incl - self_f

    for cc in range(0, C, CCHUNK):
        c_iota = (jax.lax.broadcasted_iota(jnp.int32, (1, CCHUNK), 1)
                  + cc).astype(jnp.float32)
        le = (cum_incl <= c_iota).astype(jnp.float32)
        topi_ref[0, 0, pl.ds(cc, CCHUNK)] = (
            jnp.sum(le, axis=0) + jnp.float32(b * N)).astype(jnp.int32)
        onehot = jnp.logical_and(slot == c_iota, sel).astype(jnp.float32)
        gval_ref[0, 0, pl.ds(cc, CCHUNK)] = jnp.sum(onehot * gate, axis=0)

    n_iota = jax.lax.broadcasted_iota(jnp.int32, (N, 1), 0)
    sentinel = BC + (n_iota & (C - 1))
    inv = jnp.where(sel, slot.astype(jnp.int32) + b * C, sentinel)
    inv_ref[0] = inv


def _router_topk(x, w):
    return pl.pallas_call(
        _router_topk_body,
        grid=(B,),
        in_specs=[
            pl.BlockSpec((1, N, D), lambda b: (b, 0, 0)),
            pl.BlockSpec((D, 1), lambda b: (0, 0)),
        ],
        out_specs=[
            pl.BlockSpec((1, 1, C), lambda b: (b, 0, 0)),
            pl.BlockSpec((1, N, 1), lambda b: (b, 0, 0)),
            pl.BlockSpec((1, 1, C), lambda b: (b, 0, 0)),
            pl.BlockSpec((1, 1, 1), lambda b: (b, 0, 0)),
        ],
        out_shape=[
            jax.ShapeDtypeStruct((B, 1, C), jnp.int32),
            jax.ShapeDtypeStruct((B, N, 1), jnp.int32),
            jax.ShapeDtypeStruct((B, 1, C), jnp.float32),
            jax.ShapeDtypeStruct((B, 1, 1), jnp.float32),
        ],
    )(x, w)


def _qkv_body(x_ref, w_ref, o_ref):
    o_ref[...] = lax.dot_general(
        x_ref[...].astype(jnp.bfloat16), w_ref[...].astype(jnp.bfloat16),
        (((1,), (0,)), ((), ())),
        preferred_element_type=jnp.float32)


def _qkv(gathered, wqkv):
    return pl.pallas_call(
        _qkv_body,
        grid=(B, 6),
        in_specs=[
            pl.BlockSpec((C, D), lambda b, j: (b, 0)),
            pl.BlockSpec((D, 512), lambda b, j: (0, j)),
        ],
        out_specs=pl.BlockSpec((C, 512), lambda b, j: (b, j)),
        out_shape=jax.ShapeDtypeStruct((BC, 3 * D), jnp.float32),
        compiler_params=pltpu.CompilerParams(
            dimension_semantics=("parallel", "parallel")),
    )(gathered, wqkv)


def _attn_body(q_ref, k_ref, v_ref, wo_ref, g_ref, o_ref):
    b = pl.program_id(0)
    j = pl.program_id(1)

    @pl.when(b < B)
    def _compute():
        scale = jnp.float32(1.0 / (DH ** 0.5))
        part = None
        for h in (0, 1):
            q = (q_ref[:, pl.ds(h * DH, DH)] * scale).astype(jnp.bfloat16)
            k = k_ref[:, pl.ds(h * DH, DH)].astype(jnp.bfloat16)
            v = v_ref[:, pl.ds(h * DH, DH)].astype(jnp.bfloat16)
            s = lax.dot_general(q, k, (((1,), (1,)), ((), ())),
                                preferred_element_type=jnp.float32)
            p = jnp.exp(s)
            denom = jnp.sum(p, axis=1, keepdims=True)
            o = lax.dot_general(p.astype(jnp.bfloat16), v,
                                (((1,), (0,)), ((), ())),
                                preferred_element_type=jnp.float32)
            o = (o / denom).astype(jnp.bfloat16)
            wo_h = wo_ref[pl.ds(h * DH, DH), :].astype(jnp.bfloat16)
            ph = lax.dot_general(o, wo_h, (((1,), (0,)), ((), ())),
                                 preferred_element_type=jnp.float32)
            part = ph if part is None else part + ph

        @pl.when(j == 0)
        def _():
            o_ref[...] = part

        @pl.when(j > 0)
        def _():
            o_ref[...] = o_ref[...] + part

        @pl.when(j == H // 2 - 1)
        def _():
            gcol = jnp.transpose(g_ref[0])
            o_ref[...] = o_ref[...] * gcol

    @pl.when(jnp.logical_and(b == B, j == 0))
    def _zero():
        o_ref[...] = jnp.zeros_like(o_ref)


def _attn(qkv, wo, gval):
    cb = lambda b: jnp.minimum(b, B - 1)
    return pl.pallas_call(
        _attn_body,
        grid=(B + 1, H // 2),
        in_specs=[
            pl.BlockSpec((C, 128), lambda b, j: (cb(b), j)),
            pl.BlockSpec((C, 128), lambda b, j: (cb(b), 8 + j)),
            pl.BlockSpec((C, 128), lambda b, j: (cb(b), 16 + j)),
            pl.BlockSpec((128, D), lambda b, j: (j, 0)),
            pl.BlockSpec((1, 1, C), lambda b, j: (cb(b), 0, 0)),
        ],
        out_specs=pl.BlockSpec((C, D), lambda b, j: (b, 0)),
        out_shape=jax.ShapeDtypeStruct(((B + 1) * C, D), jnp.float32),
        compiler_params=pltpu.CompilerParams(
            dimension_semantics=("parallel", "arbitrary")),
    )(qkv, qkv, qkv, wo, gval)


def _make_sc_gather(n_rows_out, chunk):
    info = plsc.get_sparse_core_info()
    nw = info.num_cores * info.num_subcores
    per_w = n_rows_out // nw
    n_chunks = per_w // chunk
    mesh = plsc.VectorSubcoreMesh(core_axis_name="c", subcore_axis_name="s")

    @functools.partial(
        pl.kernel,
        mesh=mesh,
        out_type=jax.ShapeDtypeStruct((n_rows_out, D), jnp.float32),
        scratch_types=[
            pltpu.VMEM((chunk,), jnp.int32),
            pltpu.VMEM((chunk, D), jnp.float32),
            pltpu.SemaphoreType.DMA,
        ],
    )
    def _gather(table_hbm, idx_hbm, out_hbm, idx_v, rows_v, sem):
        wid = lax.axis_index("s") * info.num_cores + lax.axis_index("c")
        base = wid * per_w
        for ch in range(n_chunks):
            off = base + ch * chunk
            pltpu.sync_copy(idx_hbm.at[pl.ds(off, chunk)], idx_v)
            pltpu.async_copy(table_hbm.at[idx_v], rows_v, sem).wait()
            pltpu.sync_copy(rows_v, out_hbm.at[pl.ds(off, chunk)])

    return _gather


def kernel(token_inputs, W_router, Wqkv, Wo):
    topi, inv, gval, zsum = _router_topk(token_inputs, W_router)

    x2 = token_inputs.reshape(BN, D)
    gathered = _make_sc_gather(BC, 64)(x2, topi.reshape(BC))

    qkv = _qkv(gathered, Wqkv)
    gated = _attn(qkv, Wo, gval)

    out2 = _make_sc_gather(BN, 64)(gated, inv.reshape(BN))
    output = out2.reshape(B, N, D)

    z_loss = jnp.sum(zsum) / jnp.float32(B * N)
    return (output, z_loss)


# speedup vs baseline: 1.3594x; 1.3594x over previous
import functools

import jax
import jax.numpy as jnp
from jax import lax
from jax.experimental import pallas as pl
from jax.experimental.pallas import tpu as pltpu
from jax.experimental.pallas import tpu_sc as plsc

B, N, D = 4, 2048, 1024
H, DH = 16, 64
C = N // 2
BC = B * C
BN = B * N


def _cumsum_row(v):
    n = v.shape[1]
    s = 1
    while s < n:
        shifted = jnp.concatenate(
            [jnp.zeros((1, s), jnp.float32), lax.slice(v, (0, 0), (1, n - s))],
            axis=1)
        v = v + shifted
        s *= 2
    return v


def _router_topk_body(x_ref, w_ref, topi_ref, inv_ref, gval_ref, zsum_ref):
    b = pl.program_id(0)
    x = x_ref[0]
    w = w_ref[...]
    logits = lax.dot_general(w, x, (((0,), (1,)), ((), ())),
                             preferred_element_type=jnp.float32)
    zsum_ref[...] = jnp.sum(logits * logits, keepdims=True).reshape(1, 1, 1)
    gate = jax.nn.sigmoid(logits)

    lc = jnp.where(logits == 0.0, 0.0, logits)
    u = lax.bitcast_convert_type(lc, jnp.uint32)
    mask = jnp.where((u >> jnp.uint32(31)) > jnp.uint32(0),
                     jnp.uint32(0xFFFFFFFF), jnp.uint32(0x80000000))
    key = u ^ mask

    t = jnp.zeros((1, 1), jnp.uint32)
    cf = jnp.float32(C)
    for bit in range(31, -1, -1):
        cand = t | jnp.uint32(1 << bit)
        cnt = jnp.sum((key >= cand).astype(jnp.float32))
        t = jnp.where(cnt >= cf, cand, t)

    gt = key > t
    eq = key == t
    cnt_gt = jnp.sum(gt.astype(jnp.float32))
    need = cf - cnt_gt
    eqf = eq.astype(jnp.float32)
    eq_excl = _cumsum_row(eqf) - eqf
    sel = jnp.logical_or(gt, jnp.logical_and(eq, eq_excl < need))
    sel_f = sel.astype(jnp.float32)
    cum_incl = _cumsum_row(sel_f)
    slot = cum_incl - sel_f

    slotm = jnp.where(sel, slot, -1.0)
    n_iota = jax.lax.broadcasted_iota(jnp.int32, (1, N), 1).astype(jnp.float32)
    g2 = jnp.concatenate([n_iota, gate], axis=0)
    c_col = jax.lax.broadcasted_iota(jnp.int32, (C, 1), 0).astype(jnp.float32)
    eqc = (slotm == c_col).astype(jnp.float32)
    res = lax.dot_general(eqc, g2, (((1,), (1,)), ((), ())),
                          preferred_element_type=jnp.float32)
    topi_ref[0] = (res[:, 0:1] + jnp.float32(0.5)).astype(jnp.int32) + b * N
    gval_ref[0] = res[:, 1:2]

    sentinel = BC + (jax.lax.broadcasted_iota(jnp.int32, (1, N), 1) & (C - 1))
    inv_ref[0] = jnp.where(sel, slot.astype(jnp.int32) + b * C, sentinel)


def _router_topk(x, w):
    return pl.pallas_call(
        _router_topk_body,
        grid=(B,),
        in_specs=[
            pl.BlockSpec((1, N, D), lambda b: (b, 0, 0)),
            pl.BlockSpec((D, 1), lambda b: (0, 0)),
        ],
        out_specs=[
            pl.BlockSpec((1, C, 1), lambda b: (b, 0, 0)),
            pl.BlockSpec((1, 1, N), lambda b: (b, 0, 0)),
            pl.BlockSpec((1, C, 1), lambda b: (b, 0, 0)),
            pl.BlockSpec((1, 1, 1), lambda b: (b, 0, 0)),
        ],
        out_shape=[
            jax.ShapeDtypeStruct((B, C, 1), jnp.int32),
            jax.ShapeDtypeStruct((B, 1, N), jnp.int32),
            jax.ShapeDtypeStruct((B, C, 1), jnp.float32),
            jax.ShapeDtypeStruct((B, 1, 1), jnp.float32),
        ],
    )(x, w)


def _qkv_body(x_ref, w_ref, o_ref):
    o_ref[...] = lax.dot_general(
        x_ref[...].astype(jnp.bfloat16), w_ref[...].astype(jnp.bfloat16),
        (((1,), (0,)), ((), ())),
        preferred_element_type=jnp.float32)


def _qkv(gathered, wqkv):
    return pl.pallas_call(
        _qkv_body,
        grid=(B, 6),
        in_specs=[
            pl.BlockSpec((C, D), lambda b, j: (b, 0)),
            pl.BlockSpec((D, 512), lambda b, j: (0, j)),
        ],
        out_specs=pl.BlockSpec((C, 512), lambda b, j: (b, j)),
        out_shape=jax.ShapeDtypeStruct((BC, 3 * D), jnp.float32),
        compiler_params=pltpu.CompilerParams(
            dimension_semantics=("parallel", "parallel")),
    )(gathered, wqkv)


def _attn_body(q_ref, k_ref, v_ref, o_ref):
    scale = jnp.float32(1.0 / (DH ** 0.5))
    for h in (0, 1):
        q = (q_ref[:, pl.ds(h * DH, DH)] * scale).astype(jnp.bfloat16)
        k = k_ref[:, pl.ds(h * DH, DH)].astype(jnp.bfloat16)
        v = v_ref[:, pl.ds(h * DH, DH)].astype(jnp.bfloat16)
        s = lax.dot_general(q, k, (((1,), (1,)), ((), ())),
                            preferred_element_type=jnp.float32)
        p = jnp.exp(s)
        denom = jnp.sum(p, axis=1, keepdims=True)
        o = lax.dot_general(p.astype(jnp.bfloat16), v,
                            (((1,), (0,)), ((), ())),
                            preferred_element_type=jnp.float32)
        o_ref[:, pl.ds(h * DH, DH)] = o / denom


def _attn(qkv):
    return pl.pallas_call(
        _attn_body,
        grid=(B, H // 2),
        in_specs=[
            pl.BlockSpec((C, 128), lambda b, j: (b, j)),
            pl.BlockSpec((C, 128), lambda b, j: (b, 8 + j)),
            pl.BlockSpec((C, 128), lambda b, j: (b, 16 + j)),
        ],
        out_specs=pl.BlockSpec((C, 128), lambda b, j: (b, j)),
        out_shape=jax.ShapeDtypeStruct((BC, D), jnp.float32),
        compiler_params=pltpu.CompilerParams(
            dimension_semantics=("parallel", "parallel")),
    )(qkv, qkv, qkv)


def _wo_body(a_ref, w_ref, g_ref, o_ref):
    b = pl.program_id(0)

    @pl.when(b < B)
    def _():
        o_ref[...] = lax.dot_general(
            a_ref[...].astype(jnp.bfloat16), w_ref[...].astype(jnp.bfloat16),
            (((1,), (0,)), ((), ())),
            preferred_element_type=jnp.float32) * g_ref[0]

    @pl.when(b == B)
    def _zero():
        o_ref[...] = jnp.zeros_like(o_ref)


def _wo(attn_cat, wo, gval):
    cb = lambda b: jnp.minimum(b, B - 1)
    return pl.pallas_call(
        _wo_body,
        grid=(B + 1, 2),
        in_specs=[
            pl.BlockSpec((C, D), lambda b, j: (cb(b), 0)),
            pl.BlockSpec((D, 512), lambda b, j: (0, j)),
            pl.BlockSpec((1, C, 1), lambda b, j: (cb(b), 0, 0)),
        ],
        out_specs=pl.BlockSpec((C, 512), lambda b, j: (b, j)),
        out_shape=jax.ShapeDtypeStruct(((B + 1) * C, D), jnp.float32),
        compiler_params=pltpu.CompilerParams(
            dimension_semantics=("parallel", "parallel")),
    )(attn_cat, wo, gval)


def _make_sc_gather(n_rows_out, chunk):
    info = plsc.get_sparse_core_info()
    nw = info.num_cores * info.num_subcores
    per_w = n_rows_out // nw
    n_chunks = per_w // chunk
    mesh = plsc.VectorSubcoreMesh(core_axis_name="c", subcore_axis_name="s")

    @functools.partial(
        pl.kernel,
        mesh=mesh,
        out_type=jax.ShapeDtypeStruct((n_rows_out, D), jnp.float32),
        scratch_types=[
            pltpu.VMEM((chunk,), jnp.int32),
            pltpu.VMEM((chunk, D), jnp.float32),
            pltpu.SemaphoreType.DMA,
        ],
    )
    def _gather(table_hbm, idx_hbm, out_hbm, idx_v, rows_v, sem):
        wid = lax.axis_index("s") * info.num_cores + lax.axis_index("c")
        base = wid * per_w
        for ch in range(n_chunks):
            off = base + ch * chunk
            pltpu.sync_copy(idx_hbm.at[pl.ds(off, chunk)], idx_v)
            pltpu.async_copy(table_hbm.at[idx_v], rows_v, sem).wait()
            pltpu.sync_copy(rows_v, out_hbm.at[pl.ds(off, chunk)])

    return _gather


def kernel(token_inputs, W_router, Wqkv, Wo):
    topi, inv, gval, zsum = _router_topk(token_inputs, W_router)

    x2 = token_inputs.reshape(BN, D)
    gathered = _make_sc_gather(BC, 64)(x2, topi.reshape(BC))

    qkv = _qkv(gathered, Wqkv)
    attn_cat = _attn(qkv)
    gated = _wo(attn_cat, Wo, gval)

    out2 = _make_sc_gather(BN, 64)(gated, inv.reshape(BN))
    output = out2.reshape(B, N, D)

    z_loss = jnp.sum(zsum) / jnp.float32(B * N)
    return (output, z_loss)


# speedup vs baseline: 1.3965x; 1.0273x over previous
import functools

import jax
import jax.numpy as jnp
from jax import lax
from jax.experimental import pallas as pl
from jax.experimental.pallas import tpu as pltpu
from jax.experimental.pallas import tpu_sc as plsc

B, N, D = 4, 2048, 1024
H, DH = 16, 64
C = N // 2
BC = B * C
BN = B * N


def _cumsum_row(v):
    n = v.shape[1]
    s = 1
    while s < n:
        shifted = jnp.concatenate(
            [jnp.zeros((1, s), jnp.float32), lax.slice(v, (0, 0), (1, n - s))],
            axis=1)
        v = v + shifted
        s *= 2
    return v


def _router_topk_body(x_ref, w_ref, topi_ref, inv_ref, gval_ref, zsum_ref):
    b = pl.program_id(0)
    x = x_ref[0]
    w = w_ref[...]
    logits = lax.dot_general(w, x, (((0,), (1,)), ((), ())),
                             preferred_element_type=jnp.float32)
    zsum_ref[...] = jnp.sum(logits * logits, keepdims=True).reshape(1, 1, 1)
    gate = jax.nn.sigmoid(logits)

    lc = jnp.where(logits == 0.0, 0.0, logits)
    u = lax.bitcast_convert_type(lc, jnp.uint32)
    mask = jnp.where((u >> jnp.uint32(31)) > jnp.uint32(0),
                     jnp.uint32(0xFFFFFFFF), jnp.uint32(0x80000000))
    key = u ^ mask

    t = jnp.zeros((1, 1), jnp.uint32)
    cf = jnp.float32(C)
    for bit in range(31, -1, -1):
        cand = t | jnp.uint32(1 << bit)
        cnt = jnp.sum((key >= cand).astype(jnp.float32))
        t = jnp.where(cnt >= cf, cand, t)

    gt = key > t
    eq = key == t
    cnt_gt = jnp.sum(gt.astype(jnp.float32))
    need = cf - cnt_gt
    eqf = eq.astype(jnp.float32)
    eq_excl = _cumsum_row(eqf) - eqf
    sel = jnp.logical_or(gt, jnp.logical_and(eq, eq_excl < need))
    sel_f = sel.astype(jnp.float32)
    cum_incl = _cumsum_row(sel_f)
    slot = cum_incl - sel_f

    slotm = jnp.where(sel, slot, -1.0)
    n_iota = jax.lax.broadcasted_iota(jnp.int32, (1, N), 1).astype(jnp.float32)
    g2 = jnp.concatenate([n_iota, gate], axis=0)
    c_col = jax.lax.broadcasted_iota(jnp.int32, (C, 1), 0).astype(jnp.float32)
    eqc = (slotm == c_col).astype(jnp.float32)
    res = lax.dot_general(eqc, g2, (((1,), (1,)), ((), ())),
                          preferred_element_type=jnp.float32)
    topi_ref[0] = (res[:, 0:1] + jnp.float32(0.5)).astype(jnp.int32) + b * N
    gval_ref[0] = res[:, 1:2]

    sentinel = BC + (jax.lax.broadcasted_iota(jnp.int32, (1, N), 1) & (C - 1))
    inv_ref[0] = jnp.where(sel, slot.astype(jnp.int32) + b * C, sentinel)


def _router_topk(x, w):
    return pl.pallas_call(
        _router_topk_body,
        grid=(B,),
        in_specs=[
            pl.BlockSpec((1, N, D), lambda b: (b, 0, 0)),
            pl.BlockSpec((D, 1), lambda b: (0, 0)),
        ],
        out_specs=[
            pl.BlockSpec((1, C, 1), lambda b: (b, 0, 0)),
            pl.BlockSpec((1, 1, N), lambda b: (b, 0, 0)),
            pl.BlockSpec((1, C, 1), lambda b: (b, 0, 0)),
            pl.BlockSpec((1, 1, 1), lambda b: (b, 0, 0)),
        ],
        out_shape=[
            jax.ShapeDtypeStruct((B, C, 1), jnp.int32),
            jax.ShapeDtypeStruct((B, 1, N), jnp.int32),
            jax.ShapeDtypeStruct((B, C, 1), jnp.float32),
            jax.ShapeDtypeStruct((B, 1, 1), jnp.float32),
        ],
    )(x, w)


def _qkv_body(x_ref, w_ref, o_ref):
    o_ref[...] = lax.dot_general(
        x_ref[...].astype(jnp.bfloat16), w_ref[...].astype(jnp.bfloat16),
        (((1,), (0,)), ((), ())),
        preferred_element_type=jnp.float32).astype(jnp.bfloat16)


def _qkv(gathered, wqkv):
    return pl.pallas_call(
        _qkv_body,
        grid=(B, 6),
        in_specs=[
            pl.BlockSpec((C, D), lambda b, j: (b, 0)),
            pl.BlockSpec((D, 512), lambda b, j: (0, j)),
        ],
        out_specs=pl.BlockSpec((C, 512), lambda b, j: (b, j)),
        out_shape=jax.ShapeDtypeStruct((BC, 3 * D), jnp.bfloat16),
        compiler_params=pltpu.CompilerParams(
            dimension_semantics=("parallel", "parallel")),
    )(gathered, wqkv)


def _attn_body(q_ref, k_ref, v_ref, o_ref):
    scale = jnp.bfloat16(1.0 / (DH ** 0.5))
    for h in (0, 1):
        q = q_ref[:, pl.ds(h * DH, DH)] * scale
        k = k_ref[:, pl.ds(h * DH, DH)]
        v = v_ref[:, pl.ds(h * DH, DH)]
        s = lax.dot_general(q, k, (((1,), (1,)), ((), ())),
                            preferred_element_type=jnp.float32)
        p = jnp.exp(s)
        denom = jnp.sum(p, axis=1, keepdims=True)
        o = lax.dot_general(p.astype(jnp.bfloat16), v,
                            (((1,), (0,)), ((), ())),
                            preferred_element_type=jnp.float32)
        o_ref[:, pl.ds(h * DH, DH)] = (o / denom).astype(jnp.bfloat16)


def _attn(qkv):
    return pl.pallas_call(
        _attn_body,
        grid=(B, H // 2),
        in_specs=[
            pl.BlockSpec((C, 128), lambda b, j: (b, j)),
            pl.BlockSpec((C, 128), lambda b, j: (b, 8 + j)),
            pl.BlockSpec((C, 128), lambda b, j: (b, 16 + j)),
        ],
        out_specs=pl.BlockSpec((C, 128), lambda b, j: (b, j)),
        out_shape=jax.ShapeDtypeStruct((BC, D), jnp.bfloat16),
        compiler_params=pltpu.CompilerParams(
            dimension_semantics=("parallel", "parallel")),
    )(qkv, qkv, qkv)


def _wo_body(a_ref, w_ref, g_ref, o_ref):
    b = pl.program_id(0)

    @pl.when(b < B)
    def _():
        o_ref[...] = lax.dot_general(
            a_ref[...], w_ref[...].astype(jnp.bfloat16),
            (((1,), (0,)), ((), ())),
            preferred_element_type=jnp.float32) * g_ref[0]

    @pl.when(b == B)
    def _zero():
        o_ref[...] = jnp.zeros_like(o_ref)


def _wo(attn_cat, wo, gval):
    cb = lambda b: jnp.minimum(b, B - 1)
    return pl.pallas_call(
        _wo_body,
        grid=(B + 1, 2),
        in_specs=[
            pl.BlockSpec((C, D), lambda b, j: (cb(b), 0)),
            pl.BlockSpec((D, 512), lambda b, j: (0, j)),
            pl.BlockSpec((1, C, 1), lambda b, j: (cb(b), 0, 0)),
        ],
        out_specs=pl.BlockSpec((C, 512), lambda b, j: (b, j)),
        out_shape=jax.ShapeDtypeStruct(((B + 1) * C, D), jnp.float32),
        compiler_params=pltpu.CompilerParams(
            dimension_semantics=("parallel", "parallel")),
    )(attn_cat, wo, gval)


def _make_sc_gather(n_rows_out, chunk):
    info = plsc.get_sparse_core_info()
    nw = info.num_cores * info.num_subcores
    per_w = n_rows_out // nw
    n_chunks = per_w // chunk
    mesh = plsc.VectorSubcoreMesh(core_axis_name="c", subcore_axis_name="s")

    @functools.partial(
        pl.kernel,
        mesh=mesh,
        out_type=jax.ShapeDtypeStruct((n_rows_out, D), jnp.float32),
        scratch_types=[
            pltpu.VMEM((chunk,), jnp.int32),
            pltpu.VMEM((chunk, D), jnp.float32),
            pltpu.SemaphoreType.DMA,
        ],
    )
    def _gather(table_hbm, idx_hbm, out_hbm, idx_v, rows_v, sem):
        wid = lax.axis_index("s") * info.num_cores + lax.axis_index("c")
        base = wid * per_w
        for ch in range(n_chunks):
            off = base + ch * chunk
            pltpu.sync_copy(idx_hbm.at[pl.ds(off, chunk)], idx_v)
            pltpu.async_copy(table_hbm.at[idx_v], rows_v, sem).wait()
            pltpu.sync_copy(rows_v, out_hbm.at[pl.ds(off, chunk)])

    return _gather


def kernel(token_inputs, W_router, Wqkv, Wo):
    topi, inv, gval, zsum = _router_topk(token_inputs, W_router)

    x2 = token_inputs.reshape(BN, D)
    gathered = _make_sc_gather(BC, 64)(x2, topi.reshape(BC))

    qkv = _qkv(gathered, Wqkv)
    attn_cat = _attn(qkv)
    gated = _wo(attn_cat, Wo, gval)

    out2 = _make_sc_gather(BN, 64)(gated, inv.reshape(BN))
    output = out2.reshape(B, N, D)

    z_loss = jnp.sum(zsum) / jnp.float32(B * N)
    return (output, z_loss)


# speedup vs baseline: 1.4339x; 1.0268x over previous
import functools

import jax
import jax.numpy as jnp
from jax import lax
from jax.experimental import pallas as pl
from jax.experimental.pallas import tpu as pltpu
from jax.experimental.pallas import tpu_sc as plsc

B, N, D = 4, 2048, 1024
H, DH = 16, 64
C = N // 2
BC = B * C
BN = B * N


def _cumsum_row(v):
    n = v.shape[1]
    s = 1
    while s < n:
        shifted = jnp.concatenate(
            [jnp.zeros((1, s), jnp.float32), lax.slice(v, (0, 0), (1, n - s))],
            axis=1)
        v = v + shifted
        s *= 2
    return v


def _router_topk_body(x_ref, w_ref, topi_ref, inv_ref, gval_ref, zsum_ref):
    b = pl.program_id(0)
    x = x_ref[0]
    w = w_ref[...]
    logits = lax.dot_general(w, x, (((0,), (1,)), ((), ())),
                             preferred_element_type=jnp.float32)
    zsum_ref[...] = jnp.sum(logits * logits, keepdims=True).reshape(1, 1, 1)
    gate = jax.nn.sigmoid(logits)

    lc = jnp.where(logits == 0.0, 0.0, logits)
    u = lax.bitcast_convert_type(lc, jnp.uint32)
    mask = jnp.where((u >> jnp.uint32(31)) > jnp.uint32(0),
                     jnp.uint32(0xFFFFFFFF), jnp.uint32(0x80000000))
    key = u ^ mask

    t = jnp.zeros((1, 1), jnp.uint32)
    cf = jnp.float32(C)
    for bit in range(30, -1, -2):
        c3 = t | jnp.uint32(3 << bit)
        c2 = t | jnp.uint32(2 << bit)
        c1 = t | jnp.uint32(1 << bit)
        n3 = jnp.sum((key >= c3).astype(jnp.float32))
        n2 = jnp.sum((key >= c2).astype(jnp.float32))
        n1 = jnp.sum((key >= c1).astype(jnp.float32))
        t = jnp.where(n3 >= cf, c3,
                      jnp.where(n2 >= cf, c2, jnp.where(n1 >= cf, c1, t)))

    gt = key > t
    eq = key == t
    cnt_gt = jnp.sum(gt.astype(jnp.float32))
    need = cf - cnt_gt
    eqf = eq.astype(jnp.float32)
    eq_excl = _cumsum_row(eqf) - eqf
    sel = jnp.logical_or(gt, jnp.logical_and(eq, eq_excl < need))
    sel_f = sel.astype(jnp.float32)
    cum_incl = _cumsum_row(sel_f)
    slot = cum_incl - sel_f

    slotm = jnp.where(sel, slot, -1.0)
    n_iota = jax.lax.broadcasted_iota(jnp.int32, (1, N), 1).astype(jnp.float32)
    g2 = jnp.concatenate([n_iota, gate], axis=0)
    c_col = jax.lax.broadcasted_iota(jnp.int32, (C, 1), 0).astype(jnp.float32)
    eqc = (slotm == c_col).astype(jnp.float32)
    res = lax.dot_general(eqc, g2, (((1,), (1,)), ((), ())),
                          preferred_element_type=jnp.float32)
    topi_ref[0] = (res[:, 0:1] + jnp.float32(0.5)).astype(jnp.int32) + b * N
    gval_ref[0] = res[:, 1:2]

    sentinel = BC + (jax.lax.broadcasted_iota(jnp.int32, (1, N), 1) & (C - 1))
    inv_ref[0] = jnp.where(sel, slot.astype(jnp.int32) + b * C, sentinel)


def _router_topk(x, w):
    return pl.pallas_call(
        _router_topk_body,
        grid=(B,),
        in_specs=[
            pl.BlockSpec((1, N, D), lambda b: (b, 0, 0)),
            pl.BlockSpec((D, 1), lambda b: (0, 0)),
        ],
        out_specs=[
            pl.BlockSpec((1, C, 1), lambda b: (b, 0, 0)),
            pl.BlockSpec((1, 1, N), lambda b: (b, 0, 0)),
            pl.BlockSpec((1, C, 1), lambda b: (b, 0, 0)),
            pl.BlockSpec((1, 1, 1), lambda b: (b, 0, 0)),
        ],
        out_shape=[
            jax.ShapeDtypeStruct((B, C, 1), jnp.int32),
            jax.ShapeDtypeStruct((B, 1, N), jnp.int32),
            jax.ShapeDtypeStruct((B, C, 1), jnp.float32),
            jax.ShapeDtypeStruct((B, 1, 1), jnp.float32),
        ],
    )(x, w)


def _qkv_body(x_ref, w_ref, o_ref):
    o_ref[...] = lax.dot_general(
        x_ref[...].astype(jnp.bfloat16), w_ref[...].astype(jnp.bfloat16),
        (((1,), (0,)), ((), ())),
        preferred_element_type=jnp.float32).astype(jnp.bfloat16)


def _qkv(gathered, wqkv):
    return pl.pallas_call(
        _qkv_body,
        grid=(B, 6),
        in_specs=[
            pl.BlockSpec((C, D), lambda b, j: (b, 0)),
            pl.BlockSpec((D, 512), lambda b, j: (0, j)),
        ],
        out_specs=pl.BlockSpec((C, 512), lambda b, j: (b, j)),
        out_shape=jax.ShapeDtypeStruct((BC, 3 * D), jnp.bfloat16),
        compiler_params=pltpu.CompilerParams(
            dimension_semantics=("parallel", "parallel")),
    )(gathered, wqkv)


def _attn_body(q_ref, k_ref, v_ref, o_ref):
    scale = jnp.bfloat16(1.0 / (DH ** 0.5))
    for h in (0, 1):
        q = q_ref[:, pl.ds(h * DH, DH)] * scale
        k = k_ref[:, pl.ds(h * DH, DH)]
        v = v_ref[:, pl.ds(h * DH, DH)]
        s = lax.dot_general(q, k, (((1,), (1,)), ((), ())),
                            preferred_element_type=jnp.float32)
        p = jnp.exp(s)
        denom = jnp.sum(p, axis=1, keepdims=True)
        o = lax.dot_general(p.astype(jnp.bfloat16), v,
                            (((1,), (0,)), ((), ())),
                            preferred_element_type=jnp.float32)
        o_ref[:, pl.ds(h * DH, DH)] = (o / denom).astype(jnp.bfloat16)


def _attn(qkv):
    return pl.pallas_call(
        _attn_body,
        grid=(B, H // 2),
        in_specs=[
            pl.BlockSpec((C, 128), lambda b, j: (b, j)),
            pl.BlockSpec((C, 128), lambda b, j: (b, 8 + j)),
            pl.BlockSpec((C, 128), lambda b, j: (b, 16 + j)),
        ],
        out_specs=pl.BlockSpec((C, 128), lambda b, j: (b, j)),
        out_shape=jax.ShapeDtypeStruct((BC, D), jnp.bfloat16),
        compiler_params=pltpu.CompilerParams(
            dimension_semantics=("parallel", "parallel")),
    )(qkv, qkv, qkv)


def _wo_body(a_ref, w_ref, g_ref, o_ref):
    b = pl.program_id(0)

    @pl.when(b < B)
    def _():
        o_ref[...] = lax.dot_general(
            a_ref[...], w_ref[...].astype(jnp.bfloat16),
            (((1,), (0,)), ((), ())),
            preferred_element_type=jnp.float32) * g_ref[0]

    @pl.when(b == B)
    def _zero():
        o_ref[...] = jnp.zeros_like(o_ref)


def _wo(attn_cat, wo, gval):
    cb = lambda b: jnp.minimum(b, B - 1)
    return pl.pallas_call(
        _wo_body,
        grid=(B + 1, 2),
        in_specs=[
            pl.BlockSpec((C, D), lambda b, j: (cb(b), 0)),
            pl.BlockSpec((D, 512), lambda b, j: (0, j)),
            pl.BlockSpec((1, C, 1), lambda b, j: (cb(b), 0, 0)),
        ],
        out_specs=pl.BlockSpec((C, 512), lambda b, j: (b, j)),
        out_shape=jax.ShapeDtypeStruct(((B + 1) * C, D), jnp.float32),
        compiler_params=pltpu.CompilerParams(
            dimension_semantics=("parallel", "parallel")),
    )(attn_cat, wo, gval)


def _make_sc_gather(n_rows_out, chunk):
    info = plsc.get_sparse_core_info()
    nw = info.num_cores * info.num_subcores
    per_w = n_rows_out // nw
    n_chunks = per_w // chunk
    mesh = plsc.VectorSubcoreMesh(core_axis_name="c", subcore_axis_name="s")

    @functools.partial(
        pl.kernel,
        mesh=mesh,
        out_type=jax.ShapeDtypeStruct((n_rows_out, D), jnp.float32),
        scratch_types=[
            pltpu.VMEM((chunk,), jnp.int32),
            pltpu.VMEM((chunk, D), jnp.float32),
            pltpu.SemaphoreType.DMA,
        ],
    )
    def _gather(table_hbm, idx_hbm, out_hbm, idx_v, rows_v, sem):
        wid = lax.axis_index("s") * info.num_cores + lax.axis_index("c")
        base = wid * per_w
        for ch in range(n_chunks):
            off = base + ch * chunk
            pltpu.sync_copy(idx_hbm.at[pl.ds(off, chunk)], idx_v)
            pltpu.async_copy(table_hbm.at[idx_v], rows_v, sem).wait()
            pltpu.sync_copy(rows_v, out_hbm.at[pl.ds(off, chunk)])

    return _gather


def kernel(token_inputs, W_router, Wqkv, Wo):
    topi, inv, gval, zsum = _router_topk(token_inputs, W_router)

    x2 = token_inputs.reshape(BN, D)
    gathered = _make_sc_gather(BC, 64)(x2, topi.reshape(BC))

    qkv = _qkv(gathered, Wqkv)
    attn_cat = _attn(qkv)
    gated = _wo(attn_cat, Wo, gval)

    out2 = _make_sc_gather(BN, 64)(gated, inv.reshape(BN))
    output = out2.reshape(B, N, D)

    z_loss = jnp.sum(zsum) / jnp.float32(B * N)
    return (output, z_loss)
